# baseline (speedup 1.0000x reference)
import functools

import jax
import jax.numpy as jnp
from jax import lax
from jax.experimental import pallas as pl
from jax.experimental.pallas import tpu as pltpu
from jax.experimental.pallas import tpu_sc as plsc

_K = 8
_EPS = 1e-8
_NEG = -1e30


def _topk_body(tok_rows_ref, tok_all_ref, out_ref):
    rows = tok_rows_ref[...]
    allt = tok_all_ref[...]
    rn = rows / (jnp.sqrt(jnp.sum(rows * rows, axis=1, keepdims=True)) + _EPS)
    an = allt / (jnp.sqrt(jnp.sum(allt * allt, axis=1, keepdims=True)) + _EPS)
    sim = lax.dot_general(rn, an, (((1,), (1,)), ((), ())),
                          preferred_element_type=jnp.float32)
    RT, N = sim.shape
    r = pl.program_id(0)
    col = lax.broadcasted_iota(jnp.int32, (RT, N), 1)
    row_glob = lax.broadcasted_iota(jnp.int32, (RT, N), 0) + r * RT
    sim = jnp.where(col == row_glob, _NEG, sim)
    idxs = []
    for _ in range(_K):
        idx = jnp.argmax(sim, axis=1).astype(jnp.int32)[:, None]
        idxs.append(idx)
        sim = jnp.where(col == idx, _NEG, sim)
    out_ref[...] = jnp.concatenate(idxs, axis=1)


def _topk_tc(tokens_b, rt=1024, interpret=False):
    N, D = tokens_b.shape
    return pl.pallas_call(
        _topk_body,
        grid=(N // rt,),
        in_specs=[
            pl.BlockSpec((rt, D), lambda r: (r, 0)),
            pl.BlockSpec((N, D), lambda r: (0, 0)),
        ],
        out_specs=pl.BlockSpec((rt, _K), lambda r: (r, 0)),
        out_shape=jax.ShapeDtypeStruct((N, _K), jnp.int32),
        interpret=interpret,
    )(tokens_b, tokens_b)


@functools.cache
def _adj_sc_call(N):
    info = plsc.get_sparse_core_info()
    NC, NS, L = info.num_cores, info.num_subcores, info.num_lanes
    NW = NC * NS
    rows_w = N // NW
    RIT = 16
    n_it = rows_w // RIT
    mesh = plsc.VectorSubcoreMesh(core_axis_name="c", subcore_axis_name="s")

    @functools.partial(
        pl.kernel,
        out_type=jax.ShapeDtypeStruct((N * N,), jnp.float32),
        mesh=mesh,
        scratch_types=[
            pltpu.VMEM((N * _K,), jnp.int32),
            pltpu.VMEM((RIT * N,), jnp.float32),
        ],
        compiler_params=pltpu.CompilerParams(needs_layout_passes=False),
    )
    def adj_kernel(topk_hbm, out_hbm, tbl, rowbuf):
        wid = lax.axis_index("s") * NC + lax.axis_index("c")
        row0 = wid * rows_w

        pltpu.sync_copy(topk_hbm, tbl)

        zero = jnp.zeros((L,), jnp.float32)

        def _zb(i, carry):
            rowbuf[pl.ds(i * L, L)] = zero
            return carry

        lax.fori_loop(0, (RIT * N) // L, _zb, 0)

        lane = lax.iota(jnp.int32, 16)
        second = (lane >= _K).astype(jnp.int32)
        ones = jnp.ones((16,), jnp.float32)

        def _chunk(it, carry):
            row_b = row0 + it * RIT
            for v in range(RIT // 2):
                r0 = row_b + 2 * v
                nbrs = tbl[pl.ds(r0 * _K, 2 * _K)]
                rowvec = jnp.full((16,), r0, jnp.int32) + second
                acc = lane < 0
                for l in range(_K):
                    g = plsc.load_gather(tbl, [nbrs * _K + l])
                    acc = jnp.logical_or(acc, g == rowvec)
                scat = (2 * v + second) * N + nbrs
                plsc.store_scatter(rowbuf, [scat], ones, mask=acc)
            pltpu.sync_copy(rowbuf, out_hbm.at[pl.ds(row_b * N, RIT * N)])
            for v in range(RIT // 2):
                r0 = row_b + 2 * v
                nbrs = tbl[pl.ds(r0 * _K, 2 * _K)]
                scat = (2 * v + second) * N + nbrs
                plsc.store_scatter(rowbuf, [scat], zero)
            return carry

        lax.fori_loop(0, n_it, _chunk, 0)

    return adj_kernel


def kernel(tokens):
    B, N, D = tokens.shape
    adj_call = _adj_sc_call(N)
    adjs = []
    for b in range(B):
        topk_b = _topk_tc(tokens[b])
        adjs.append(adj_call(topk_b.reshape(N * _K)))
    return jnp.stack(adjs).reshape(B, N, N)


# speedup vs baseline: 1.1187x; 1.1187x over previous
import functools

import jax
import jax.numpy as jnp
from jax import lax
from jax.experimental import pallas as pl
from jax.experimental.pallas import tpu as pltpu
from jax.experimental.pallas import tpu_sc as plsc

_K = 8
_EPS = 1e-8
_NEG = -1e30


def _topk_body(tok_rows_ref, tok_all_ref, out_ref):
    rn = tok_rows_ref[0].astype(jnp.bfloat16)
    an = tok_all_ref[0].astype(jnp.bfloat16)
    sim = lax.dot_general(rn, an, (((1,), (1,)), ((), ())),
                          preferred_element_type=jnp.float32)
    RT, N = sim.shape
    r = pl.program_id(1)
    col = lax.broadcasted_iota(jnp.int32, (RT, N), 1)
    row_glob = lax.broadcasted_iota(jnp.int32, (RT, N), 0) + r * RT
    sim = jnp.where(col == row_glob, _NEG, sim)
    idxs = []
    for _ in range(_K):
        idx = jnp.argmax(sim, axis=1).astype(jnp.int32)[:, None]
        idxs.append(idx)
        sim = jnp.where(col == idx, _NEG, sim)
    out_ref[0] = jnp.concatenate(idxs, axis=1)


def _topk_tc(tokens, rt=512, interpret=False):
    B, N, D = tokens.shape
    grid = (B, N // rt)
    return pl.pallas_call(
        _topk_body,
        grid=grid,
        in_specs=[
            pl.BlockSpec((1, rt, D), lambda b, r: (b, r, 0)),
            pl.BlockSpec((1, N, D), lambda b, r: (b, 0, 0)),
        ],
        out_specs=pl.BlockSpec((1, rt, _K), lambda b, r: (b, r, 0)),
        out_shape=jax.ShapeDtypeStruct((B, N, _K), jnp.int32),
        interpret=interpret,
    )(tokens, tokens)


def _adj_sc(topk, B, N):
    info = plsc.get_sparse_core_info()
    NC, NS, L = info.num_cores, info.num_subcores, info.num_lanes
    NW = NC * NS
    rows_w = (B * N) // NW
    RIT = 16
    n_it = rows_w // RIT
    w_per_b = N // rows_w
    mesh = plsc.VectorSubcoreMesh(core_axis_name="c", subcore_axis_name="s")

    @functools.partial(
        pl.kernel,
        out_type=jax.ShapeDtypeStruct((B * N * N,), jnp.float32),
        mesh=mesh,
        scratch_types=[
            pltpu.VMEM((N * _K,), jnp.int32),
            pltpu.VMEM((RIT * N,), jnp.float32),
        ],
        compiler_params=pltpu.CompilerParams(needs_layout_passes=False),
    )
    def adj_kernel(topk_hbm, out_hbm, tbl, rowbuf):
        wid = lax.axis_index("s") * NC + lax.axis_index("c")
        b = wid // w_per_b
        row0_b = (wid % w_per_b) * rows_w

        pltpu.sync_copy(topk_hbm.at[b], tbl)

        zero = jnp.zeros((L,), jnp.float32)

        def _zb(i, carry):
            rowbuf[pl.ds(i * L, L)] = zero
            return carry

        lax.fori_loop(0, (RIT * N) // L, _zb, 0)

        lane = lax.iota(jnp.int32, 16)
        second = (lane >= _K).astype(jnp.int32)
        ones = jnp.ones((16,), jnp.float32)

        def _chunk(it, carry):
            row_b = row0_b + it * RIT
            for v in range(RIT // 2):
                r0 = row_b + 2 * v
                nbrs = tbl[pl.ds(r0 * _K, 2 * _K)]
                rowvec = jnp.full((16,), r0, jnp.int32) + second
                acc = lane < 0
                for l in range(_K):
                    g = plsc.load_gather(tbl, [nbrs * _K + l])
                    acc = jnp.logical_or(acc, g == rowvec)
                scat = (2 * v + second) * N + nbrs
                plsc.store_scatter(rowbuf, [scat], ones, mask=acc)
            out0 = (b * N + row_b) * N
            pltpu.sync_copy(rowbuf, out_hbm.at[pl.ds(out0, RIT * N)])
            for v in range(RIT // 2):
                r0 = row_b + 2 * v
                nbrs = tbl[pl.ds(r0 * _K, 2 * _K)]
                scat = (2 * v + second) * N + nbrs
                plsc.store_scatter(rowbuf, [scat], zero)
            return carry

        lax.fori_loop(0, n_it, _chunk, 0)

    return adj_kernel(topk)


def kernel(tokens):
    B, N, D = tokens.shape
    tokens = tokens / (jnp.linalg.norm(tokens, axis=-1, keepdims=True) + _EPS)
    topk = _topk_tc(tokens)
    adj = _adj_sc(topk.reshape(B, N * _K), B, N)
    return adj.reshape(B, N, N)
